# Initial kernel scaffold; baseline (speedup 1.0000x reference)
import jax
import jax.numpy as jnp
from jax.experimental import pallas as pl


def kernel(x, edge_index, edge_type, W1, W2):
    raise NotImplementedError("write your pallas kernel here")


# speedup vs baseline: 11.1575x; 11.1575x over previous
import functools

import jax
import jax.numpy as jnp
from jax import lax
from jax.experimental import pallas as pl
from jax.experimental.pallas import tpu as pltpu
from jax.experimental.pallas import tpu_sc as plsc

N = 10000
E = 320000
R = 8
D_IN = 128
D_H = 128
C = 40
C_PAD = 128

NC = 2
NS = 16
NW = NC * NS

NRP = 81920
K = 80
EPT = E // NW
NB_T = EPT // K
EPS = E // NS
NB_C = EPS // K

N_ACC = 10240
ROWS_PER_TILE = N_ACC // NS


def _mesh():
    return plsc.VectorSubcoreMesh(core_axis_name="c", subcore_axis_name="s")


def _make_count_kernel():
    S = NRP // NS

    @functools.partial(
        pl.kernel,
        out_type=jax.ShapeDtypeStruct((E,), jnp.float32),
        mesh=_mesh(),
        scratch_types=[
            pltpu.VMEM((K,), jnp.int32),
            pltpu.VMEM((K,), jnp.int32),
            pltpu.VMEM((K,), jnp.int32),
            pltpu.VMEM((K,), jnp.float32),
            pltpu.VMEM((K,), jnp.float32),
            pltpu.VMEM((S,), jnp.float32),
            pltpu.VMEM_SHARED((NRP,), jnp.float32),
        ],
    )
    def count_kernel(dst_hbm, rel_hbm, w_hbm,
                     dst_v, rel_v, idx_v, ones_v, w_v, cbuf, counts_sh):
        cid = lax.axis_index("c")
        sid = lax.axis_index("s")
        wid = sid * NC + cid

        zero16 = jnp.zeros((16,), jnp.float32)
        one16 = jnp.ones((16,), jnp.float32)

        def zb(i, carry):
            cbuf[pl.ds(i * 16, 16)] = zero16
            return carry

        lax.fori_loop(0, S // 16, zb, 0)
        for i in range(K // 16):
            ones_v[pl.ds(i * 16, 16)] = one16
        pltpu.sync_copy(cbuf, counts_sh.at[pl.ds(sid * S, S)])
        plsc.subcore_barrier()

        def cb(b, carry):
            base = sid * EPS + b * K
            pltpu.sync_copy(dst_hbm.at[pl.ds(base, K)], dst_v)
            pltpu.sync_copy(rel_hbm.at[pl.ds(base, K)], rel_v)
            for i in range(K // 16):
                sl = pl.ds(i * 16, 16)
                idx_v[sl] = dst_v[sl] * R + rel_v[sl]
            pltpu.sync_copy(ones_v, counts_sh.at[idx_v], add=True)
            return carry

        lax.fori_loop(0, NB_C, cb, 0)
        plsc.subcore_barrier()

        pltpu.sync_copy(counts_sh.at[pl.ds(sid * S, S)], cbuf)

        def rb(i, carry):
            sl = pl.ds(i * 16, 16)
            cbuf[sl] = one16 / jnp.maximum(cbuf[sl], one16)
            return carry

        lax.fori_loop(0, S // 16, rb, 0)
        pltpu.sync_copy(cbuf, counts_sh.at[pl.ds(sid * S, S)])
        plsc.subcore_barrier()

        def wb(b, carry):
            base = wid * EPT + b * K
            pltpu.sync_copy(dst_hbm.at[pl.ds(base, K)], dst_v)
            pltpu.sync_copy(rel_hbm.at[pl.ds(base, K)], rel_v)
            for i in range(K // 16):
                sl = pl.ds(i * 16, 16)
                idx_v[sl] = dst_v[sl] * R + rel_v[sl]
            pltpu.sync_copy(counts_sh.at[idx_v], w_v)
            pltpu.sync_copy(w_v, w_hbm.at[pl.ds(base, K)])
            return carry

        lax.fori_loop(0, NB_T, wb, 0)

    return count_kernel


def _make_agg_kernel(D):
    @functools.partial(
        pl.kernel,
        out_type=jax.ShapeDtypeStruct((2 * N_ACC, D), jnp.float32),
        mesh=_mesh(),
        scratch_types=[
            pltpu.VMEM((K,), jnp.int32),
            pltpu.VMEM((K,), jnp.int32),
            pltpu.VMEM((K,), jnp.int32),
            pltpu.VMEM((K,), jnp.float32),
            pltpu.VMEM((K,), jnp.int32),
            pltpu.VMEM((K, D), jnp.float32),
            pltpu.VMEM((128, D), jnp.float32),
            pltpu.VMEM_SHARED((N_ACC, D), jnp.float32),
            pltpu.SemaphoreType.DMA,
        ],
    )
    def agg_kernel(z_hbm, src_hbm, rel_hbm, dst_hbm, w_hbm, out_hbm,
                   src_v, rel_v, dst_v, w_v, gidx_v, rows_v, zbuf, acc_sh,
                   sem):
        cid = lax.axis_index("c")
        sid = lax.axis_index("s")
        wid = sid * NC + cid

        zero16 = jnp.zeros((16,), jnp.float32)

        def zr(i, carry):
            for j in range(D // 16):
                zbuf[i, pl.ds(j * 16, 16)] = zero16
            return carry

        lax.fori_loop(0, 128, zr, 0)
        for q in range(ROWS_PER_TILE // 128):
            pltpu.sync_copy(
                zbuf, acc_sh.at[pl.ds(sid * ROWS_PER_TILE + q * 128, 128)])
        plsc.subcore_barrier()

        def bb(b, carry):
            base = wid * EPT + b * K
            pltpu.sync_copy(src_hbm.at[pl.ds(base, K)], src_v)
            pltpu.sync_copy(rel_hbm.at[pl.ds(base, K)], rel_v)
            pltpu.sync_copy(dst_hbm.at[pl.ds(base, K)], dst_v)
            pltpu.sync_copy(w_hbm.at[pl.ds(base, K)], w_v)
            for i in range(K // 16):
                sl = pl.ds(i * 16, 16)
                gidx_v[sl] = rel_v[sl] * N + src_v[sl]
            pltpu.async_copy(z_hbm.at[gidx_v], rows_v, sem).wait()

            def sb(g, carry2):
                w16 = w_v[pl.ds(g * 16, 16)]
                for l in range(16):
                    wv = w16.at[jnp.full((16,), l, jnp.int32)].get(
                        mode="promise_in_bounds")
                    i = g * 16 + l
                    for j in range(D // 16):
                        sl2 = pl.ds(j * 16, 16)
                        rows_v[i, sl2] = rows_v[i, sl2] * wv
                return carry2

            lax.fori_loop(0, K // 16, sb, 0)
            pltpu.sync_copy(rows_v, acc_sh.at[dst_v], add=True)
            return carry

        lax.fori_loop(0, NB_T, bb, 0)
        plsc.subcore_barrier()

        rb = sid * ROWS_PER_TILE
        pltpu.sync_copy(acc_sh.at[pl.ds(rb, ROWS_PER_TILE)],
                        out_hbm.at[pl.ds(cid * N_ACC + rb, ROWS_PER_TILE)])

    return agg_kernel


_count_kernel = _make_count_kernel()
_agg_kernel_h = _make_agg_kernel(D_H)
_agg_kernel_c = _agg_kernel_h


_NBLK = 10
_BN = N // _NBLK


def _m1_body(x_ref, w_ref, o_ref):
    o_ref[0] = jnp.dot(x_ref[...], w_ref[0],
                       preferred_element_type=jnp.float32)


def _relmm1(x, W1):
    return pl.pallas_call(
        _m1_body,
        grid=(R, _NBLK),
        in_specs=[
            pl.BlockSpec((_BN, D_IN), lambda r, i: (i, 0)),
            pl.BlockSpec((1, D_IN, D_H), lambda r, i: (r, 0, 0)),
        ],
        out_specs=pl.BlockSpec((1, _BN, D_H), lambda r, i: (r, i, 0)),
        out_shape=jax.ShapeDtypeStruct((R, N, D_H), jnp.float32),
    )(x, W1)


def _m2_body(p_ref, w_ref, o_ref):
    h = jnp.tanh(p_ref[0] + p_ref[1])
    o_ref[0] = jnp.dot(h, w_ref[0], preferred_element_type=jnp.float32)


def _relmm2(p1, W2p):
    return pl.pallas_call(
        _m2_body,
        grid=(R, _NBLK),
        in_specs=[
            pl.BlockSpec((2, _BN, D_H), lambda r, i: (0, i, 0)),
            pl.BlockSpec((1, D_H, C_PAD), lambda r, i: (r, 0, 0)),
        ],
        out_specs=pl.BlockSpec((1, _BN, C_PAD), lambda r, i: (r, i, 0)),
        out_shape=jax.ShapeDtypeStruct((R, N, C_PAD), jnp.float32),
    )(p1, W2p)


def _fin_body(p_ref, sm_ref, h_ref):
    h2 = p_ref[0] + p_ref[1]
    col = lax.broadcasted_iota(jnp.int32, h2.shape, 1)
    hm = jnp.where(col < C, h2, jnp.float32(-1e30))
    m = jnp.max(hm, axis=1, keepdims=True)
    ex = jnp.exp(hm - m)
    s = jnp.sum(ex, axis=1, keepdims=True)
    sm_ref[...] = ex / s
    h_ref[...] = h2


def _finalize(p2):
    return pl.pallas_call(
        _fin_body,
        grid=(_NBLK,),
        in_specs=[pl.BlockSpec((2, _BN, C_PAD), lambda i: (0, i, 0))],
        out_specs=[
            pl.BlockSpec((_BN, C_PAD), lambda i: (i, 0)),
            pl.BlockSpec((_BN, C_PAD), lambda i: (i, 0)),
        ],
        out_shape=[
            jax.ShapeDtypeStruct((N, C_PAD), jnp.float32),
            jax.ShapeDtypeStruct((N, C_PAD), jnp.float32),
        ],
    )(p2)


def kernel(x, edge_index, edge_type, W1, W2):
    src = edge_index[0]
    dst = edge_index[1]
    rel = edge_type

    w_edge = _count_kernel(dst, rel)

    z1 = _relmm1(x, W1).reshape(R * N, D_H)
    p1 = _agg_kernel_h(z1, src, rel, dst, w_edge)
    p1 = p1.reshape(2, N_ACC, D_H)[:, :N]

    W2p = jnp.pad(W2, ((0, 0), (0, 0), (0, C_PAD - C)))
    z2 = _relmm2(p1, W2p).reshape(R * N, C_PAD)
    p2 = _agg_kernel_c(z2, src, rel, dst, w_edge)
    p2 = p2.reshape(2, N_ACC, C_PAD)[:, :N]

    smp, h2p = _finalize(p2)
    return (smp[:, :C], h2p[:, :C])
